# Initial kernel scaffold; baseline (speedup 1.0000x reference)
import jax
import jax.numpy as jnp
from jax.experimental import pallas as pl


def kernel(query_text, answer_text, local_entity, q2e_adj_mat, kb_fact_rel, match_entity_one_hop, only_two_entity, match_entity_only_two, one_two_triples_id, posts_length, responses_length, word_embed, entity_embed, lstm_Wih, lstm_Whh, lstm_b, entity_W, entity_b):
    raise NotImplementedError("write your pallas kernel here")


# speedup vs baseline: 1.0422x; 1.0422x over previous
import functools

import jax
import jax.numpy as jnp
from jax import lax
from jax.experimental import pallas as pl
from jax.experimental.pallas import tpu as pltpu
from jax.experimental.pallas import tpu_sc as plsc

B = 128
ENC_LEN = 50
MAX_LOCAL = 512
EMBED_UNITS = 300
TRANS_UNITS = 100
W_PAD = 384
E_PAD = 128

NC = 2
NS = 16
NW = NC * NS

W_ROWS = B * ENC_LEN
W_PER = W_ROWS // NW
E_ROWS = B * MAX_LOCAL
E_PER = E_ROWS // NW
E_CHUNK = 256


def _gather_body(widxa_hbm, widxb_hbm, eidx_hbm, wtab_hbm, etab_hbm,
                 wout_hbm, eout_hbm,
                 widxa_v, widxb_v, wrows_v, eidx_v, erows_v, sem):
    wid = lax.axis_index("s") * NC + lax.axis_index("c")
    wbase = wid * W_PER
    pltpu.sync_copy(widxa_hbm.at[wid], widxa_v)
    pltpu.sync_copy(widxb_hbm.at[wid], widxb_v)
    cps = [
        pltpu.async_copy(wtab_hbm.at[widxa_v],
                         wrows_v.at[pl.ds(0, 128)], sem),
        pltpu.async_copy(wtab_hbm.at[widxb_v],
                         wrows_v.at[pl.ds(128, 72)], sem),
    ]
    for cp in cps:
        cp.wait()
    pltpu.sync_copy(wrows_v, wout_hbm.at[pl.ds(wbase, W_PER)])
    ebase = wid * E_PER
    pltpu.sync_copy(eidx_hbm.at[wid], eidx_v)
    for i in range(E_PER // E_CHUNK):
        cps = [
            pltpu.async_copy(etab_hbm.at[eidx_v.at[i * 2 + j]],
                             erows_v.at[pl.ds(j * 128, 128)], sem)
            for j in range(2)
        ]
        for cp in cps:
            cp.wait()
        pltpu.sync_copy(erows_v,
                        eout_hbm.at[pl.ds(ebase + i * E_CHUNK, E_CHUNK)])


_gather = functools.partial(
    pl.kernel,
    out_type=(
        jax.ShapeDtypeStruct((W_ROWS, W_PAD), jnp.float32),
        jax.ShapeDtypeStruct((E_ROWS, E_PAD), jnp.float32),
    ),
    mesh=plsc.VectorSubcoreMesh(core_axis_name="c", subcore_axis_name="s"),
    scratch_types=[
        pltpu.VMEM((128,), jnp.int32),
        pltpu.VMEM((72,), jnp.int32),
        pltpu.VMEM((W_PER, W_PAD), jnp.float32),
        pltpu.VMEM((16, 128), jnp.int32),
        pltpu.VMEM((E_CHUNK, E_PAD), jnp.float32),
        pltpu.SemaphoreType.DMA,
    ],
)(_gather_body)


def _lstm_body(xw_ref, wih_ref, whh_ref, b_ref, out_ref, xp_ref, h_ref, c_ref):
    xp_ref[...] = jnp.dot(xw_ref[...], wih_ref[...],
                          preferred_element_type=jnp.float32)
    h_ref[...] = jnp.zeros_like(h_ref)
    c_ref[...] = jnp.zeros_like(c_ref)

    def step(t, _):
        g = (xp_ref[pl.ds(t * B, B), :]
             + jnp.dot(h_ref[...], whh_ref[...],
                       preferred_element_type=jnp.float32)
             + b_ref[...])
        i = jax.nn.sigmoid(g[:, 0:128])
        f = jax.nn.sigmoid(g[:, 128:256])
        gg = jnp.tanh(g[:, 256:384])
        o = jax.nn.sigmoid(g[:, 384:512])
        c = f * c_ref[...] + i * gg
        h = o * jnp.tanh(c)
        c_ref[...] = c
        h_ref[...] = h
        out_ref[pl.ds(t * B, B), :] = h
        return 0

    lax.fori_loop(0, ENC_LEN, step, 0)


def _entlin_body(rows_ref, w_ref, b_ref, out_ref):
    out_ref[...] = jnp.maximum(
        jnp.dot(rows_ref[...], w_ref[...],
                preferred_element_type=jnp.float32) + b_ref[...], 0.0)


def kernel(query_text, answer_text, local_entity, q2e_adj_mat, kb_fact_rel,
           match_entity_one_hop, only_two_entity, match_entity_only_two,
           one_two_triples_id, posts_length, responses_length,
           word_embed, entity_embed, lstm_Wih, lstm_Whh, lstm_b,
           entity_W, entity_b):
    local_entity_mask = (local_entity != 0).astype(jnp.float32)
    query_mask = (query_text != 0).astype(jnp.float32)
    pagerank_f = q2e_adj_mat
    responses_id = jnp.concatenate(
        [jnp.ones((B, 1), answer_text.dtype), answer_text[:, :-1]], axis=1)

    wtab = jnp.pad(word_embed, ((0, 0), (0, W_PAD - EMBED_UNITS)))
    etab = jnp.pad(entity_embed, ((0, 0), (0, E_PAD - TRANS_UNITS)))

    widx = jnp.transpose(query_text).reshape(NW, W_PER)
    widxa = widx[:, :128]
    widxb = widx[:, 128:]
    eidx = local_entity.reshape(NW, 16, 128)
    wrows, erows = _gather(widxa, widxb, eidx, wtab, etab)

    wih_p = jnp.pad(lstm_Wih.T.reshape(EMBED_UNITS, 4, TRANS_UNITS),
                    ((0, W_PAD - EMBED_UNITS), (0, 0), (0, 28))
                    ).reshape(W_PAD, 512)
    whh_p = jnp.pad(lstm_Whh.T.reshape(TRANS_UNITS, 4, TRANS_UNITS),
                    ((0, 28), (0, 0), (0, 28))).reshape(128, 512)
    b_p = jnp.pad(lstm_b.reshape(4, TRANS_UNITS),
                  ((0, 0), (0, 28))).reshape(1, 512)

    hs = pl.pallas_call(
        _lstm_body,
        out_shape=jax.ShapeDtypeStruct((W_ROWS, 128), jnp.float32),
        scratch_shapes=[
            pltpu.VMEM((W_ROWS, 512), jnp.float32),
            pltpu.VMEM((B, 128), jnp.float32),
            pltpu.VMEM((B, 128), jnp.float32),
        ],
    )(wrows, wih_p, whh_p, b_p)
    hs = hs[:, :TRANS_UNITS].reshape(ENC_LEN, B, TRANS_UNITS)
    query_hidden_emb = jnp.transpose(hs, (1, 0, 2))
    query_node_emb = hs[-1][None]

    w_p = jnp.pad(entity_W.T, ((0, E_PAD - TRANS_UNITS), (0, 0)))
    blk = 4096
    ent = pl.pallas_call(
        _entlin_body,
        grid=(E_ROWS // blk,),
        in_specs=[
            pl.BlockSpec((blk, E_PAD), lambda i: (i, 0)),
            pl.BlockSpec((E_PAD, TRANS_UNITS), lambda i: (0, 0)),
            pl.BlockSpec((1, TRANS_UNITS), lambda i: (0, 0)),
        ],
        out_specs=pl.BlockSpec((blk, TRANS_UNITS), lambda i: (i, 0)),
        out_shape=jax.ShapeDtypeStruct((E_ROWS, TRANS_UNITS), jnp.float32),
    )(erows, w_p, entity_b[None])
    local_entity_emb = ent.reshape(B, MAX_LOCAL, TRANS_UNITS)

    return (query_hidden_emb, query_node_emb, local_entity_emb,
            local_entity_mask, query_mask, responses_id, pagerank_f)


# speedup vs baseline: 1.9633x; 1.8839x over previous
import functools

import jax
import jax.numpy as jnp
from jax import lax
from jax.experimental import pallas as pl
from jax.experimental.pallas import tpu as pltpu
from jax.experimental.pallas import tpu_sc as plsc

B = 128
ENC_LEN = 50
MAX_LOCAL = 512
EMBED_UNITS = 300
TRANS_UNITS = 100
WORD_VOCAB = 30000
ENT_VOCAB = 100007
W_PAD = 384
E_PAD = 128

NC = 2
NS = 16
NW = NC * NS

W_ROWS = B * ENC_LEN
W_PER = W_ROWS // NW
E_ROWS = B * MAX_LOCAL
E_PER = E_ROWS // NW
E_CHUNK = 512


def _padw_body(src_ref, out_ref):
    out_ref[...] = jnp.concatenate(
        [src_ref[...],
         jnp.zeros((src_ref.shape[0], W_PAD - EMBED_UNITS), jnp.float32)],
        axis=1)


def _pade_body(src_ref, out_ref):
    out_ref[...] = jnp.concatenate(
        [src_ref[...],
         jnp.zeros((src_ref.shape[0], E_PAD - TRANS_UNITS), jnp.float32)],
        axis=1)


def _gather_word_body(widxa_hbm, widxb_hbm, wtab_hbm, wout_hbm,
                      widxa_v, widxb_v, wrows_v, sem):
    wid = lax.axis_index("s") * NC + lax.axis_index("c")
    wbase = wid * W_PER
    pltpu.sync_copy(widxa_hbm.at[wid], widxa_v)
    pltpu.sync_copy(widxb_hbm.at[wid], widxb_v)
    cps = [
        pltpu.async_copy(wtab_hbm.at[widxa_v],
                         wrows_v.at[pl.ds(0, 128)], sem),
        pltpu.async_copy(wtab_hbm.at[widxb_v],
                         wrows_v.at[pl.ds(128, 72)], sem),
    ]
    for cp in cps:
        cp.wait()
    pltpu.sync_copy(wrows_v, wout_hbm.at[pl.ds(wbase, W_PER)])


_gather_word = functools.partial(
    pl.kernel,
    out_type=jax.ShapeDtypeStruct((W_ROWS, W_PAD), jnp.float32),
    mesh=plsc.VectorSubcoreMesh(core_axis_name="c", subcore_axis_name="s"),
    scratch_types=[
        pltpu.VMEM((128,), jnp.int32),
        pltpu.VMEM((72,), jnp.int32),
        pltpu.VMEM((W_PER, W_PAD), jnp.float32),
        pltpu.SemaphoreType.DMA,
    ],
)(_gather_word_body)


def _gather_ent_body(eidx_hbm, etab_hbm, eout_hbm, eidx_v, erows_v, sem):
    wid = lax.axis_index("s") * NC + lax.axis_index("c")
    ebase = wid * E_PER
    pltpu.sync_copy(eidx_hbm.at[wid], eidx_v)
    for i in range(E_PER // E_CHUNK):
        cps = [
            pltpu.async_copy(etab_hbm.at[eidx_v.at[i * 4 + j]],
                             erows_v.at[pl.ds(j * 128, 128)], sem)
            for j in range(4)
        ]
        for cp in cps:
            cp.wait()
        pltpu.sync_copy(erows_v,
                        eout_hbm.at[pl.ds(ebase + i * E_CHUNK, E_CHUNK)])


_gather_ent = functools.partial(
    pl.kernel,
    out_type=jax.ShapeDtypeStruct((E_ROWS, E_PAD), jnp.float32),
    mesh=plsc.VectorSubcoreMesh(core_axis_name="c", subcore_axis_name="s"),
    scratch_types=[
        pltpu.VMEM((16, 128), jnp.int32),
        pltpu.VMEM((E_CHUNK, E_PAD), jnp.float32),
        pltpu.SemaphoreType.DMA,
    ],
)(_gather_ent_body)


def _lstm_body(xw_ref, wih_ref, whh_ref, b_ref, out_ref, xp_ref, h_ref, c_ref):
    xp_ref[...] = jnp.dot(xw_ref[...], wih_ref[...],
                          preferred_element_type=jnp.float32)
    h_ref[...] = jnp.zeros_like(h_ref)
    c_ref[...] = jnp.zeros_like(c_ref)

    def step(t, _):
        g = (xp_ref[pl.ds(t * B, B), :]
             + jnp.dot(h_ref[...], whh_ref[...],
                       preferred_element_type=jnp.float32)
             + b_ref[...])
        i = jax.nn.sigmoid(g[:, 0:128])
        f = jax.nn.sigmoid(g[:, 128:256])
        gg = jnp.tanh(g[:, 256:384])
        o = jax.nn.sigmoid(g[:, 384:512])
        c = f * c_ref[...] + i * gg
        h = o * jnp.tanh(c)
        c_ref[...] = c
        h_ref[...] = h
        out_ref[pl.ds(t * B, B), :] = h
        return 0

    lax.fori_loop(0, ENC_LEN, step, 0)


def _entlin_body(rows_ref, w_ref, b_ref, out_ref):
    out_ref[...] = jnp.maximum(
        jnp.dot(rows_ref[...], w_ref[...],
                preferred_element_type=jnp.float32) + b_ref[...], 0.0)


def kernel(query_text, answer_text, local_entity, q2e_adj_mat, kb_fact_rel,
           match_entity_one_hop, only_two_entity, match_entity_only_two,
           one_two_triples_id, posts_length, responses_length,
           word_embed, entity_embed, lstm_Wih, lstm_Whh, lstm_b,
           entity_W, entity_b):
    local_entity_mask = (local_entity != 0).astype(jnp.float32)
    query_mask = (query_text != 0).astype(jnp.float32)
    pagerank_f = q2e_adj_mat
    responses_id = jnp.concatenate(
        [jnp.ones((B, 1), answer_text.dtype), answer_text[:, :-1]], axis=1)

    wtab = pl.pallas_call(
        _padw_body,
        grid=(WORD_VOCAB // 1000,),
        in_specs=[pl.BlockSpec((1000, EMBED_UNITS), lambda i: (i, 0))],
        out_specs=pl.BlockSpec((1000, W_PAD), lambda i: (i, 0)),
        out_shape=jax.ShapeDtypeStruct((WORD_VOCAB, W_PAD), jnp.float32),
    )(word_embed)
    eblk = 4096
    etab = pl.pallas_call(
        _pade_body,
        grid=(pl.cdiv(ENT_VOCAB, eblk),),
        in_specs=[pl.BlockSpec((eblk, TRANS_UNITS), lambda i: (i, 0))],
        out_specs=pl.BlockSpec((eblk, E_PAD), lambda i: (i, 0)),
        out_shape=jax.ShapeDtypeStruct((ENT_VOCAB, E_PAD), jnp.float32),
    )(entity_embed)

    widx = jnp.transpose(query_text).reshape(NW, W_PER)
    widxa = widx[:, :128]
    widxb = widx[:, 128:]
    eidx = local_entity.reshape(NW, 16, 128)
    wrows = _gather_word(widxa, widxb, wtab)
    erows = _gather_ent(eidx, etab)

    wih_p = jnp.pad(lstm_Wih.T.reshape(EMBED_UNITS, 4, TRANS_UNITS),
                    ((0, W_PAD - EMBED_UNITS), (0, 0), (0, 28))
                    ).reshape(W_PAD, 512)
    whh_p = jnp.pad(lstm_Whh.T.reshape(TRANS_UNITS, 4, TRANS_UNITS),
                    ((0, 28), (0, 0), (0, 28))).reshape(128, 512)
    b_p = jnp.pad(lstm_b.reshape(4, TRANS_UNITS),
                  ((0, 0), (0, 28))).reshape(1, 512)

    hs = pl.pallas_call(
        _lstm_body,
        out_shape=jax.ShapeDtypeStruct((W_ROWS, 128), jnp.float32),
        scratch_shapes=[
            pltpu.VMEM((W_ROWS, 512), jnp.float32),
            pltpu.VMEM((B, 128), jnp.float32),
            pltpu.VMEM((B, 128), jnp.float32),
        ],
    )(wrows, wih_p, whh_p, b_p)
    hs = hs[:, :TRANS_UNITS].reshape(ENC_LEN, B, TRANS_UNITS)
    query_hidden_emb = jnp.transpose(hs, (1, 0, 2))
    query_node_emb = hs[-1][None]

    w_p = jnp.pad(entity_W.T, ((0, E_PAD - TRANS_UNITS), (0, 0)))
    blk = 4096
    ent = pl.pallas_call(
        _entlin_body,
        grid=(E_ROWS // blk,),
        in_specs=[
            pl.BlockSpec((blk, E_PAD), lambda i: (i, 0)),
            pl.BlockSpec((E_PAD, TRANS_UNITS), lambda i: (0, 0)),
            pl.BlockSpec((1, TRANS_UNITS), lambda i: (0, 0)),
        ],
        out_specs=pl.BlockSpec((blk, TRANS_UNITS), lambda i: (i, 0)),
        out_shape=jax.ShapeDtypeStruct((E_ROWS, TRANS_UNITS), jnp.float32),
    )(erows, w_p, entity_b[None])
    local_entity_emb = ent.reshape(B, MAX_LOCAL, TRANS_UNITS)

    return (query_hidden_emb, query_node_emb, local_entity_emb,
            local_entity_mask, query_mask, responses_id, pagerank_f)


# speedup vs baseline: 1.9635x; 1.0001x over previous
import functools

import jax
import jax.numpy as jnp
from jax import lax
from jax.experimental import pallas as pl
from jax.experimental.pallas import tpu as pltpu
from jax.experimental.pallas import tpu_sc as plsc

B = 128
ENC_LEN = 50
MAX_LOCAL = 512
EMBED_UNITS = 300
TRANS_UNITS = 100
WORD_VOCAB = 30000
ENT_VOCAB = 100007
W_PAD = 384
E_PAD = 128

NC = 2
NS = 16
NW = NC * NS

W_ROWS = B * ENC_LEN
W_PER = W_ROWS // NW
E_ROWS = B * MAX_LOCAL
E_PER = E_ROWS // NW
E_CHUNK = 512


def _padw_body(src_ref, out_ref):
    out_ref[...] = jnp.concatenate(
        [src_ref[...],
         jnp.zeros((src_ref.shape[0], W_PAD - EMBED_UNITS), jnp.float32)],
        axis=1)


def _pade_body(src_ref, out_ref):
    out_ref[...] = jnp.concatenate(
        [src_ref[...],
         jnp.zeros((src_ref.shape[0], E_PAD - TRANS_UNITS), jnp.float32)],
        axis=1)


def _gather_word_body(widxa_hbm, widxb_hbm, wtab_hbm, wout_hbm,
                      widxa_v, widxb_v, wrows_v, sem):
    wid = lax.axis_index("s") * NC + lax.axis_index("c")
    wbase = wid * W_PER
    pltpu.sync_copy(widxa_hbm.at[wid], widxa_v)
    pltpu.sync_copy(widxb_hbm.at[wid], widxb_v)
    cps = [
        pltpu.async_copy(wtab_hbm.at[widxa_v],
                         wrows_v.at[pl.ds(0, 128)], sem),
        pltpu.async_copy(wtab_hbm.at[widxb_v],
                         wrows_v.at[pl.ds(128, 72)], sem),
    ]
    for cp in cps:
        cp.wait()
    pltpu.sync_copy(wrows_v, wout_hbm.at[pl.ds(wbase, W_PER)])


_gather_word = functools.partial(
    pl.kernel,
    out_type=jax.ShapeDtypeStruct((W_ROWS, W_PAD), jnp.float32),
    mesh=plsc.VectorSubcoreMesh(core_axis_name="c", subcore_axis_name="s"),
    scratch_types=[
        pltpu.VMEM((128,), jnp.int32),
        pltpu.VMEM((72,), jnp.int32),
        pltpu.VMEM((W_PER, W_PAD), jnp.float32),
        pltpu.SemaphoreType.DMA,
    ],
)(_gather_word_body)


def _gather_ent_body(eidx_hbm, etab_hbm, eout_hbm, eidx_v, erows_v, sem):
    wid = lax.axis_index("s") * NC + lax.axis_index("c")
    ebase = wid * E_PER
    pltpu.sync_copy(eidx_hbm.at[wid], eidx_v)
    for i in range(E_PER // E_CHUNK):
        cps = [
            pltpu.async_copy(etab_hbm.at[eidx_v.at[i * 4 + j]],
                             erows_v.at[pl.ds(j * 128, 128)], sem)
            for j in range(4)
        ]
        for cp in cps:
            cp.wait()
        pltpu.sync_copy(erows_v,
                        eout_hbm.at[pl.ds(ebase + i * E_CHUNK, E_CHUNK)])


_gather_ent = functools.partial(
    pl.kernel,
    out_type=jax.ShapeDtypeStruct((E_ROWS, E_PAD), jnp.float32),
    mesh=plsc.VectorSubcoreMesh(core_axis_name="c", subcore_axis_name="s"),
    scratch_types=[
        pltpu.VMEM((16, 128), jnp.int32),
        pltpu.VMEM((E_CHUNK, E_PAD), jnp.float32),
        pltpu.SemaphoreType.DMA,
    ],
)(_gather_ent_body)


def _lstm_body(xw_ref, wih_ref, whh_ref, b_ref, out_ref, xp_ref, h_ref, c_ref):
    xp_ref[...] = jnp.dot(xw_ref[...], wih_ref[...],
                          preferred_element_type=jnp.float32)
    h_ref[...] = jnp.zeros_like(h_ref)
    c_ref[...] = jnp.zeros_like(c_ref)

    def step(t, _):
        g = (xp_ref[pl.ds(t * B, B), :]
             + jnp.dot(h_ref[...], whh_ref[...],
                       preferred_element_type=jnp.float32)
             + b_ref[...])
        i = jax.nn.sigmoid(g[:, 0:128])
        f = jax.nn.sigmoid(g[:, 128:256])
        gg = jnp.tanh(g[:, 256:384])
        o = jax.nn.sigmoid(g[:, 384:512])
        c = f * c_ref[...] + i * gg
        h = o * jnp.tanh(c)
        c_ref[...] = c
        h_ref[...] = h
        out_ref[pl.ds(t * B, B), :] = h
        return 0

    lax.fori_loop(0, ENC_LEN, step, 0)


def _entlin_body(rows_ref, w_ref, b_ref, out_ref):
    out_ref[...] = jnp.maximum(
        jnp.dot(rows_ref[...], w_ref[...],
                preferred_element_type=jnp.float32) + b_ref[...], 0.0)


def kernel(query_text, answer_text, local_entity, q2e_adj_mat, kb_fact_rel,
           match_entity_one_hop, only_two_entity, match_entity_only_two,
           one_two_triples_id, posts_length, responses_length,
           word_embed, entity_embed, lstm_Wih, lstm_Whh, lstm_b,
           entity_W, entity_b):
    local_entity_mask = (local_entity != 0).astype(jnp.float32)
    query_mask = (query_text != 0).astype(jnp.float32)
    pagerank_f = q2e_adj_mat
    responses_id = jnp.concatenate(
        [jnp.ones((B, 1), answer_text.dtype), answer_text[:, :-1]], axis=1)

    wtab = pl.pallas_call(
        _padw_body,
        grid=(WORD_VOCAB // 1000,),
        in_specs=[pl.BlockSpec((1000, EMBED_UNITS), lambda i: (i, 0))],
        out_specs=pl.BlockSpec((1000, W_PAD), lambda i: (i, 0)),
        out_shape=jax.ShapeDtypeStruct((WORD_VOCAB, W_PAD), jnp.float32),
        compiler_params=pltpu.CompilerParams(needs_layout_passes=True),
    )(word_embed)
    eblk = 4096
    etab = pl.pallas_call(
        _pade_body,
        grid=(pl.cdiv(ENT_VOCAB, eblk),),
        in_specs=[pl.BlockSpec((eblk, TRANS_UNITS), lambda i: (i, 0))],
        out_specs=pl.BlockSpec((eblk, E_PAD), lambda i: (i, 0)),
        out_shape=jax.ShapeDtypeStruct((ENT_VOCAB, E_PAD), jnp.float32),
        compiler_params=pltpu.CompilerParams(needs_layout_passes=True),
    )(entity_embed)

    widx = jnp.transpose(query_text).reshape(NW, W_PER)
    widxa = widx[:, :128]
    widxb = widx[:, 128:]
    eidx = local_entity.reshape(NW, 16, 128)
    wrows = _gather_word(widxa, widxb, wtab)
    erows = _gather_ent(eidx, etab)

    wih_p = jnp.pad(lstm_Wih.T.reshape(EMBED_UNITS, 4, TRANS_UNITS),
                    ((0, W_PAD - EMBED_UNITS), (0, 0), (0, 28))
                    ).reshape(W_PAD, 512)
    whh_p = jnp.pad(lstm_Whh.T.reshape(TRANS_UNITS, 4, TRANS_UNITS),
                    ((0, 28), (0, 0), (0, 28))).reshape(128, 512)
    b_p = jnp.pad(lstm_b.reshape(4, TRANS_UNITS),
                  ((0, 0), (0, 28))).reshape(1, 512)

    hs = pl.pallas_call(
        _lstm_body,
        out_shape=jax.ShapeDtypeStruct((W_ROWS, 128), jnp.float32),
        scratch_shapes=[
            pltpu.VMEM((W_ROWS, 512), jnp.float32),
            pltpu.VMEM((B, 128), jnp.float32),
            pltpu.VMEM((B, 128), jnp.float32),
        ],
    )(wrows, wih_p, whh_p, b_p)
    hs = hs[:, :TRANS_UNITS].reshape(ENC_LEN, B, TRANS_UNITS)
    query_hidden_emb = jnp.transpose(hs, (1, 0, 2))
    query_node_emb = hs[-1][None]

    w_p = jnp.pad(entity_W.T, ((0, E_PAD - TRANS_UNITS), (0, 0)))
    blk = 4096
    ent = pl.pallas_call(
        _entlin_body,
        grid=(E_ROWS // blk,),
        in_specs=[
            pl.BlockSpec((blk, E_PAD), lambda i: (i, 0)),
            pl.BlockSpec((E_PAD, TRANS_UNITS), lambda i: (0, 0)),
            pl.BlockSpec((1, TRANS_UNITS), lambda i: (0, 0)),
        ],
        out_specs=pl.BlockSpec((blk, TRANS_UNITS), lambda i: (i, 0)),
        out_shape=jax.ShapeDtypeStruct((E_ROWS, TRANS_UNITS), jnp.float32),
    )(erows, w_p, entity_b[None])
    local_entity_emb = ent.reshape(B, MAX_LOCAL, TRANS_UNITS)

    return (query_hidden_emb, query_node_emb, local_entity_emb,
            local_entity_mask, query_mask, responses_id, pagerank_f)
